# baseline (speedup 1.0000x reference)
import functools

import jax
import jax.numpy as jnp
from jax import lax
from jax.experimental import pallas as pl
from jax.experimental.pallas import tpu as pltpu
from jax.experimental.pallas import tpu_sc as plsc

USER_N = 50000
ITEM_N = 50000
NN = USER_N + ITEM_N
NE = 1600000
EMB = 64
NL = 3
LD = 16
NDB = EMB // LD
NCORES = 2
NTILES = 16
DB_PER_CORE = NDB // NCORES

SEG = 128
CHUNK = 768
NSEG = CHUNK // SEG
NCHUNK = 132
EPT = NCHUNK * CHUNK
NE_PAD = EPT * NTILES
EROWS = 3 * NSEG

NN_PAD = -(-NN // (NTILES * 8)) * (NTILES * 8)
ROWS_PT = NN_PAD // NTILES
WBC = 384


def _wb_chunks():
    out, r = [], 0
    while r < ROWS_PT:
        out.append((r, min(WBC, ROWS_PT - r)))
        r += min(WBC, ROWS_PT - r)
    return out


def _body(earr_hbm, e0_hbm, s_hbm, e1_hbm, e2_hbm,
          acc, ebufA, ebufB, gbufA, gbufB, sidxA, sidxB,
          ssem, lsem, *gsems):
    gsemsA, gsemsB = gsems[:NSEG], gsems[NSEG:]
    c = lax.axis_index("c")
    s = lax.axis_index("s")

    def fire_lin(cid, eb):
        base = (s * NCHUNK + cid) * EROWS
        pltpu.async_copy(earr_hbm.at[pl.ds(base, EROWS)], eb, lsem)

    def drain_lin(eb):
        pltpu.make_async_copy(earr_hbm.at[pl.ds(0, EROWS)], eb, lsem).wait()

    def drain_scat(gb):
        for k in range(NSEG):
            pltpu.make_async_copy(gb.at[pl.ds(k * SEG, SEG)],
                                  acc.at[pl.ds(0, SEG)], ssem).wait()

    tables = [e0_hbm, e1_hbm, e2_hbm]
    for layer in range(NL):
        e_in = tables[layer]
        e_out = tables[layer + 1] if layer + 1 < NL else None
        s_src = e0_hbm if layer == 0 else s_hbm
        @pl.loop(0, DB_PER_CORE)
        def _(j):
            dbi = c * DB_PER_CORE + j
            dbase = dbi * NN_PAD
            e_db = e_in.at[pl.ds(dbase, NN_PAD)]

            def fire_gathers(eb, gb, gsems, e_db=e_db):
                for k in range(NSEG):
                    pltpu.async_copy(e_db.at[eb.at[k]],
                                     gb.at[pl.ds(k * SEG, SEG)], gsems[k])

            def finish_chunk(eb, gb, sidx, gsems, e_db=e_db):
                for k in range(NSEG):
                    pltpu.make_async_copy(e_db.at[eb.at[k]],
                                          gb.at[pl.ds(k * SEG, SEG)],
                                          gsems[k]).wait()

                    @plsc.parallel_loop(0, SEG // LD, unroll=1)
                    def _(i):
                        vi = eb[2 * NSEG + k, pl.ds(i * LD, LD)]
                        vv = plsc.bitcast(vi, jnp.float32)
                        sidx[k, pl.ds(i * LD, LD)] = eb[NSEG + k, pl.ds(i * LD, LD)]
                        for l in range(LD):
                            bl = lax.gather(
                                vv, jnp.full((LD, 1), l, jnp.int32),
                                lax.GatherDimensionNumbers(
                                    offset_dims=(), collapsed_slice_dims=(0,),
                                    start_index_map=(0,)),
                                (1,), mode=lax.GatherScatterMode.PROMISE_IN_BOUNDS)
                            gb[k * SEG + i * LD + l, :] = (
                                gb[k * SEG + i * LD + l, :] * bl)

                    pltpu.async_copy(gb.at[pl.ds(k * SEG, SEG)],
                                     acc.at[sidx.at[k]], ssem, add=True)

            @plsc.parallel_loop(0, WBC, unroll=8)
            def _(i):
                gbufA[i, :] = jnp.zeros((LD,), jnp.float32)

            for k, sz in _wb_chunks():
                pltpu.sync_copy(gbufA.at[pl.ds(0, sz)],
                                acc.at[pl.ds(s * ROWS_PT + k, sz)])
            plsc.subcore_barrier()

            fire_lin(0, ebufA)
            drain_lin(ebufA)
            fire_gathers(ebufA, gbufA, gsemsA)
            fire_lin(1, ebufB)

            @pl.loop(0, NCHUNK, step=2)
            def _(ci):
                @pl.when(ci > 0)
                def _():
                    drain_scat(gbufB)

                drain_lin(ebufB)
                fire_gathers(ebufB, gbufB, gsemsB)
                finish_chunk(ebufA, gbufA, sidxA, gsemsA)

                @pl.when(ci + 2 < NCHUNK)
                def _():
                    fire_lin(ci + 2, ebufA)

                drain_scat(gbufA)

                @pl.when(ci + 2 < NCHUNK)
                def _():
                    drain_lin(ebufA)
                    fire_gathers(ebufA, gbufA, gsemsA)

                finish_chunk(ebufB, gbufB, sidxB, gsemsB)

                @pl.when(ci + 3 < NCHUNK)
                def _():
                    fire_lin(ci + 3, ebufB)

            drain_scat(gbufB)
            plsc.subcore_barrier()

            wchunks = _wb_chunks()

            def drain_wb(sz, e_out=e_out):
                pltpu.make_async_copy(gbufB.at[pl.ds(0, sz)],
                                      s_hbm.at[pl.ds(0, sz)], ssem).wait()
                if e_out is not None:
                    pltpu.make_async_copy(gbufA.at[pl.ds(0, sz)],
                                          e_out.at[pl.ds(0, sz)], ssem).wait()

            for wi, (r0, sz) in enumerate(wchunks):
                p = (wi % 2) * WBC
                if wi >= 1:
                    drain_wb(wchunks[wi - 1][1])
                rbase = s * ROWS_PT + r0
                pltpu.sync_copy(acc.at[pl.ds(rbase, sz)],
                                gbufA.at[pl.ds(p, sz)])
                pltpu.sync_copy(s_src.at[pl.ds(dbase + rbase, sz)],
                                gbufB.at[pl.ds(p, sz)])

                @plsc.parallel_loop(0, sz, unroll=8)
                def _(i):
                    gbufB[p + i, :] = gbufB[p + i, :] + gbufA[p + i, :]

                pltpu.async_copy(gbufB.at[pl.ds(p, sz)],
                                 s_hbm.at[pl.ds(dbase + rbase, sz)], ssem)
                if e_out is not None:
                    pltpu.async_copy(gbufA.at[pl.ds(p, sz)],
                                     e_out.at[pl.ds(dbase + rbase, sz)], ssem)
            drain_wb(wchunks[-1][1])
            plsc.subcore_barrier()


@functools.partial(
    pl.kernel,
    out_type=(
        jax.ShapeDtypeStruct((NDB * NN_PAD, LD), jnp.float32),
        jax.ShapeDtypeStruct((NDB * NN_PAD, LD), jnp.float32),
        jax.ShapeDtypeStruct((NDB * NN_PAD, LD), jnp.float32),
    ),
    mesh=plsc.VectorSubcoreMesh(core_axis_name="c", subcore_axis_name="s"),
    compiler_params=pltpu.CompilerParams(use_tc_tiling_on_sc=False, needs_layout_passes=False),
    scratch_types=(
        pltpu.VMEM_SHARED((NN_PAD, LD), jnp.float32),
        pltpu.VMEM((EROWS, SEG), jnp.int32),
        pltpu.VMEM((EROWS, SEG), jnp.int32),
        pltpu.VMEM((CHUNK, LD), jnp.float32),
        pltpu.VMEM((CHUNK, LD), jnp.float32),
        pltpu.VMEM((NSEG, SEG), jnp.int32),
        pltpu.VMEM((NSEG, SEG), jnp.int32),
        pltpu.SemaphoreType.DMA,
        pltpu.SemaphoreType.DMA,
    ) + (pltpu.SemaphoreType.DMA,) * (2 * NSEG),
)
def _spmm3(earr_hbm, e0_hbm, s_hbm, e1_hbm, e2_hbm, *scratch):
    _body(earr_hbm, e0_hbm, s_hbm, e1_hbm, e2_hbm, *scratch)


def kernel(adj_indices, adj_values, user_emb, item_emb):
    row = adj_indices[0]
    col = adj_indices[1]
    pad = NE_PAD - NE
    shape4 = (NTILES, NCHUNK, NSEG, SEG)
    col_c = jnp.pad(col, (0, pad)).reshape(shape4)
    row_c = jnp.pad(row, (0, pad)).reshape(shape4)
    val_c = lax.bitcast_convert_type(
        jnp.pad(adj_values, (0, pad)), jnp.int32).reshape(shape4)
    earr = jnp.stack([col_c, row_c, val_c], axis=2)
    earr = earr.reshape(NTILES * NCHUNK * EROWS, SEG)

    ini = jnp.concatenate([user_emb, item_emb], axis=0)
    ini = jnp.pad(ini, ((0, NN_PAD - NN), (0, 0)))
    e0 = jnp.transpose(ini.reshape(NN_PAD, NDB, LD), (1, 0, 2)).reshape(NDB * NN_PAD, LD)

    s_out, _, _ = _spmm3(earr, e0)
    out = jnp.transpose(s_out.reshape(NDB, NN_PAD, LD)[:, :NN], (1, 0, 2)).reshape(NN, EMB)
    return out[:USER_N], out[USER_N:]


# speedup vs baseline: 1.0311x; 1.0311x over previous
import functools

import jax
import jax.numpy as jnp
from jax import lax
from jax.experimental import pallas as pl
from jax.experimental.pallas import tpu as pltpu
from jax.experimental.pallas import tpu_sc as plsc

USER_N = 50000
ITEM_N = 50000
NN = USER_N + ITEM_N
NE = 1600000
EMB = 64
NL = 3
LD = 16
NDB = EMB // LD
NCORES = 2
NTILES = 16
DB_PER_CORE = NDB // NCORES

SEG = 128
CHUNK = 768
NSEG = CHUNK // SEG
NCHUNK = 132
EPT = NCHUNK * CHUNK
NE_PAD = EPT * NTILES
EROWS = 3 * NSEG

NN_PAD = -(-NN // (NTILES * 8)) * (NTILES * 8)
ROWS_PT = NN_PAD // NTILES
WBC = 384


def _wb_chunks():
    out, r = [], 0
    while r < ROWS_PT:
        out.append((r, min(WBC, ROWS_PT - r)))
        r += min(WBC, ROWS_PT - r)
    return out


def _body(earr_hbm, e0_hbm, s_hbm, e1_hbm, e2_hbm,
          acc, ebufA, ebufB, gbufA, gbufB, sidxA, sidxB,
          ssem, lsem, *gsems):
    gsemsA, gsemsB = gsems[:NSEG], gsems[NSEG:]
    c = lax.axis_index("c")
    s = lax.axis_index("s")

    def fire_lin(cid, eb):
        base = (s * NCHUNK + cid) * EROWS
        pltpu.async_copy(earr_hbm.at[pl.ds(base, EROWS)], eb, lsem)

    def drain_lin(eb):
        pltpu.make_async_copy(earr_hbm.at[pl.ds(0, EROWS)], eb, lsem).wait()

    def drain_scat(gb):
        for k in range(NSEG):
            pltpu.make_async_copy(gb.at[pl.ds(k * SEG, SEG)],
                                  acc.at[pl.ds(0, SEG)], ssem).wait()

    tables = [e0_hbm, e1_hbm, e2_hbm]
    for layer in range(NL):
        e_in = tables[layer]
        e_out = tables[layer + 1] if layer + 1 < NL else None
        s_src = e0_hbm if layer == 0 else s_hbm
        @pl.loop(0, DB_PER_CORE)
        def _(j):
            dbi = c * DB_PER_CORE + j
            dbase = dbi * NN_PAD
            e_db = e_in.at[pl.ds(dbase, NN_PAD)]

            def fire_gathers(eb, gb, gsems, e_db=e_db):
                for k in range(NSEG):
                    pltpu.async_copy(e_db.at[eb.at[k]],
                                     gb.at[pl.ds(k * SEG, SEG)], gsems[k])

            def finish_chunk(eb, gb, sidx, gsems, e_db=e_db):
                for k in range(NSEG):
                    pltpu.make_async_copy(e_db.at[eb.at[k]],
                                          gb.at[pl.ds(k * SEG, SEG)],
                                          gsems[k]).wait()

                    @plsc.parallel_loop(0, SEG // LD, unroll=1)
                    def _(i):
                        vi = eb[2 * NSEG + k, pl.ds(i * LD, LD)]
                        vv = plsc.bitcast(vi, jnp.float32)
                        sidx[k, pl.ds(i * LD, LD)] = eb[NSEG + k, pl.ds(i * LD, LD)]
                        for l in range(LD):
                            bl = lax.gather(
                                vv, jnp.full((LD, 1), l, jnp.int32),
                                lax.GatherDimensionNumbers(
                                    offset_dims=(), collapsed_slice_dims=(0,),
                                    start_index_map=(0,)),
                                (1,), mode=lax.GatherScatterMode.PROMISE_IN_BOUNDS)
                            gb[k * SEG + i * LD + l, :] = (
                                gb[k * SEG + i * LD + l, :] * bl)

                    pltpu.async_copy(gb.at[pl.ds(k * SEG, SEG)],
                                     acc.at[sidx.at[k]], ssem, add=True)

            @plsc.parallel_loop(0, WBC, unroll=8)
            def _(i):
                gbufA[i, :] = jnp.zeros((LD,), jnp.float32)

            for k, sz in _wb_chunks():
                pltpu.sync_copy(gbufA.at[pl.ds(0, sz)],
                                acc.at[pl.ds(s * ROWS_PT + k, sz)])
            plsc.subcore_barrier()

            fire_lin(0, ebufA)
            drain_lin(ebufA)
            fire_gathers(ebufA, gbufA, gsemsA)
            fire_lin(1, ebufB)

            @pl.loop(0, NCHUNK, step=2)
            def _(ci):
                @pl.when(ci > 0)
                def _():
                    drain_scat(gbufB)

                drain_lin(ebufB)
                fire_gathers(ebufB, gbufB, gsemsB)
                finish_chunk(ebufA, gbufA, sidxA, gsemsA)

                @pl.when(ci + 2 < NCHUNK)
                def _():
                    fire_lin(ci + 2, ebufA)

                drain_scat(gbufA)

                @pl.when(ci + 2 < NCHUNK)
                def _():
                    drain_lin(ebufA)
                    fire_gathers(ebufA, gbufA, gsemsA)

                finish_chunk(ebufB, gbufB, sidxB, gsemsB)

                @pl.when(ci + 3 < NCHUNK)
                def _():
                    fire_lin(ci + 3, ebufB)

            drain_scat(gbufB)
            plsc.subcore_barrier()

            wchunks = _wb_chunks()

            def drain_wb(sz, e_out=e_out):
                pltpu.make_async_copy(gbufB.at[pl.ds(0, sz)],
                                      s_hbm.at[pl.ds(0, sz)], ssem).wait()
                if e_out is not None:
                    pltpu.make_async_copy(gbufA.at[pl.ds(0, sz)],
                                          e_out.at[pl.ds(0, sz)], ssem).wait()

            psems = (lsem, gsemsA[0])
            pltpu.async_copy(
                s_src.at[pl.ds(dbase + s * ROWS_PT, wchunks[0][1])],
                gbufB.at[pl.ds(0, wchunks[0][1])], psems[0])
            for wi, (r0, sz) in enumerate(wchunks):
                p = (wi % 2) * WBC
                if wi >= 1:
                    drain_wb(wchunks[wi - 1][1])
                if wi + 1 < len(wchunks):
                    nr0, nsz = wchunks[wi + 1]
                    pltpu.async_copy(
                        s_src.at[pl.ds(dbase + s * ROWS_PT + nr0, nsz)],
                        gbufB.at[pl.ds(((wi + 1) % 2) * WBC, nsz)],
                        psems[(wi + 1) % 2])
                rbase = s * ROWS_PT + r0
                pltpu.sync_copy(acc.at[pl.ds(rbase, sz)],
                                gbufA.at[pl.ds(p, sz)])
                pltpu.make_async_copy(
                    s_src.at[pl.ds(0, sz)],
                    gbufB.at[pl.ds(p, sz)], psems[wi % 2]).wait()

                @plsc.parallel_loop(0, sz, unroll=8)
                def _(i):
                    gbufB[p + i, :] = gbufB[p + i, :] + gbufA[p + i, :]

                pltpu.async_copy(gbufB.at[pl.ds(p, sz)],
                                 s_hbm.at[pl.ds(dbase + rbase, sz)], ssem)
                if e_out is not None:
                    pltpu.async_copy(gbufA.at[pl.ds(p, sz)],
                                     e_out.at[pl.ds(dbase + rbase, sz)], ssem)
            drain_wb(wchunks[-1][1])
            plsc.subcore_barrier()


@functools.partial(
    pl.kernel,
    out_type=(
        jax.ShapeDtypeStruct((NDB * NN_PAD, LD), jnp.float32),
        jax.ShapeDtypeStruct((NDB * NN_PAD, LD), jnp.float32),
        jax.ShapeDtypeStruct((NDB * NN_PAD, LD), jnp.float32),
    ),
    mesh=plsc.VectorSubcoreMesh(core_axis_name="c", subcore_axis_name="s"),
    compiler_params=pltpu.CompilerParams(use_tc_tiling_on_sc=False, needs_layout_passes=False),
    scratch_types=(
        pltpu.VMEM_SHARED((NN_PAD, LD), jnp.float32),
        pltpu.VMEM((EROWS, SEG), jnp.int32),
        pltpu.VMEM((EROWS, SEG), jnp.int32),
        pltpu.VMEM((CHUNK, LD), jnp.float32),
        pltpu.VMEM((CHUNK, LD), jnp.float32),
        pltpu.VMEM((NSEG, SEG), jnp.int32),
        pltpu.VMEM((NSEG, SEG), jnp.int32),
        pltpu.SemaphoreType.DMA,
        pltpu.SemaphoreType.DMA,
    ) + (pltpu.SemaphoreType.DMA,) * (2 * NSEG),
)
def _spmm3(earr_hbm, e0_hbm, s_hbm, e1_hbm, e2_hbm, *scratch):
    _body(earr_hbm, e0_hbm, s_hbm, e1_hbm, e2_hbm, *scratch)


def kernel(adj_indices, adj_values, user_emb, item_emb):
    row = adj_indices[0]
    col = adj_indices[1]
    pad = NE_PAD - NE
    shape4 = (NTILES, NCHUNK, NSEG, SEG)
    col_c = jnp.pad(col, (0, pad)).reshape(shape4)
    row_c = jnp.pad(row, (0, pad)).reshape(shape4)
    val_c = lax.bitcast_convert_type(
        jnp.pad(adj_values, (0, pad)), jnp.int32).reshape(shape4)
    earr = jnp.stack([col_c, row_c, val_c], axis=2)
    earr = earr.reshape(NTILES * NCHUNK * EROWS, SEG)

    ini = jnp.concatenate([user_emb, item_emb], axis=0)
    ini = jnp.pad(ini, ((0, NN_PAD - NN), (0, 0)))
    e0 = jnp.transpose(ini.reshape(NN_PAD, NDB, LD), (1, 0, 2)).reshape(NDB * NN_PAD, LD)

    s_out, _, _ = _spmm3(earr, e0)
    out = jnp.transpose(s_out.reshape(NDB, NN_PAD, LD)[:, :NN], (1, 0, 2)).reshape(NN, EMB)
    return out[:USER_N], out[USER_N:]
